# baseline (speedup 1.0000x reference)
import functools

import jax
import jax.numpy as jnp
from jax.experimental import pallas as pl
from jax.experimental.pallas import tpu as pltpu

H = 32
EH = 64
STEPS = 3

_MSG_TILE = 1000


def _msg_body(ef_ref, hs_ref, We1_ref, be1_ref, We2_ref, be2_ref, out_ref):
    t = jnp.maximum(
        jnp.dot(ef_ref[...], We1_ref[...], preferred_element_type=jnp.float32)
        + be1_ref[...],
        0.0,
    )
    ew = (
        jnp.dot(t, We2_ref[...], preferred_element_type=jnp.float32)
        + be2_ref[...]
    )
    hs = hs_ref[...]
    acc = hs[:, 0:1] * ew[:, 0:H]
    for i in range(1, H):
        acc = acc + hs[:, i : i + 1] * ew[:, i * H : (i + 1) * H]
    out_ref[...] = acc


def _msg_pallas(edge_feats, hs, We1, be1, We2, be2):
    E = edge_feats.shape[0]
    D_BOND = edge_feats.shape[1]
    grid = E // _MSG_TILE
    return pl.pallas_call(
        _msg_body,
        grid=(grid,),
        in_specs=[
            pl.BlockSpec((_MSG_TILE, D_BOND), lambda i: (i, 0)),
            pl.BlockSpec((_MSG_TILE, H), lambda i: (i, 0)),
            pl.BlockSpec((D_BOND, EH), lambda i: (0, 0)),
            pl.BlockSpec((1, EH), lambda i: (0, 0)),
            pl.BlockSpec((EH, H * H), lambda i: (0, 0)),
            pl.BlockSpec((1, H * H), lambda i: (0, 0)),
        ],
        out_specs=pl.BlockSpec((_MSG_TILE, H), lambda i: (i, 0)),
        out_shape=jax.ShapeDtypeStruct((E, H), jnp.float32),
    )(edge_feats, hs, We1, be1.reshape(1, EH), We2, be2.reshape(1, H * H))


def kernel(node_feats, edge_feats, edge_index, node_graph_ids, Wp, bp, We1,
           be1, We2, be2, W_ih, b_ih, W_hh, b_hh, Wpe, bpe, W1, b1, W2, b2,
           Wo, bo):
    N = node_feats.shape[0]
    G = 256
    NT = Wo.shape[1]
    src = edge_index[0]
    dst = edge_index[1]

    h = jax.nn.relu(node_feats @ Wp + bp)
    hidden = h
    for _ in range(STEPS):
        hs = jnp.take(h, src, axis=0)
        msg = _msg_pallas(edge_feats, hs, We1, be1, We2, be2)
        agg = jax.ops.segment_sum(msg, dst, num_segments=N)
        m = jax.nn.relu(agg) + h
        gi = m @ W_ih.T + b_ih
        gh = hidden @ W_hh.T + b_hh
        ir, iz, i_n = jnp.split(gi, 3, axis=1)
        hr, hz, hn = jnp.split(gh, 3, axis=1)
        r = jax.nn.sigmoid(ir + hr)
        z = jax.nn.sigmoid(iz + hz)
        n = jnp.tanh(i_n + r * hn)
        hidden = (1.0 - z) * n + z * hidden
        h = hidden

    edge_emb = jax.nn.relu(edge_feats @ Wpe + bpe)
    src_msg = jnp.concatenate([jnp.take(h, src, axis=0), edge_emb], axis=1)
    node_sum = jax.ops.segment_sum(src_msg, dst, num_segments=N)
    mol = jax.ops.segment_sum(node_sum, node_graph_ids, num_segments=G)
    mol = jax.nn.softmax(mol, axis=1)
    x1 = jax.nn.relu(mol @ W1 + b1)
    emb = jax.nn.relu(x1 @ W2 + b2)
    out = emb @ Wo + bo
    logits = out.reshape(-1, NT)
    return jax.nn.sigmoid(logits)


# speedup vs baseline: 1.8196x; 1.8196x over previous
import functools

import jax
import jax.numpy as jnp
from jax.experimental import pallas as pl
from jax.experimental.pallas import tpu as pltpu

H = 32
EH = 64
STEPS = 3

_MSG_TILE = 2000


def _msg_body(ef_ref, hs_ref, We1_ref, be1_ref, We2_ref, be2_ref, R_ref,
              out_ref):
    t = jnp.maximum(
        jnp.dot(ef_ref[...], We1_ref[...], preferred_element_type=jnp.float32)
        + be1_ref[...],
        0.0,
    )
    ew = (
        jnp.dot(t.astype(jnp.bfloat16), We2_ref[...],
                preferred_element_type=jnp.float32)
        + be2_ref[...]
    )
    rep = jnp.dot(hs_ref[...].astype(jnp.bfloat16), R_ref[...],
                  preferred_element_type=jnp.float32)
    x = ew * rep
    x = x[:, :512] + x[:, 512:]
    x = x[:, :256] + x[:, 256:]
    x = x[:, :128] + x[:, 128:]
    x = x[:, :64] + x[:, 64:]
    out_ref[...] = x[:, :32] + x[:, 32:]


def _msg_pallas(edge_feats, hs, We1, be1, We2b, be2, R):
    E = edge_feats.shape[0]
    D_BOND = edge_feats.shape[1]
    grid = E // _MSG_TILE
    return pl.pallas_call(
        _msg_body,
        grid=(grid,),
        in_specs=[
            pl.BlockSpec((_MSG_TILE, D_BOND), lambda i: (i, 0)),
            pl.BlockSpec((_MSG_TILE, H), lambda i: (i, 0)),
            pl.BlockSpec((D_BOND, EH), lambda i: (0, 0)),
            pl.BlockSpec((1, EH), lambda i: (0, 0)),
            pl.BlockSpec((EH, H * H), lambda i: (0, 0)),
            pl.BlockSpec((1, H * H), lambda i: (0, 0)),
            pl.BlockSpec((H, H * H), lambda i: (0, 0)),
        ],
        out_specs=pl.BlockSpec((_MSG_TILE, H), lambda i: (i, 0)),
        out_shape=jax.ShapeDtypeStruct((E, H), jnp.float32),
    )(edge_feats, hs, We1, be1.reshape(1, EH), We2b, be2.reshape(1, H * H), R)


def kernel(node_feats, edge_feats, edge_index, node_graph_ids, Wp, bp, We1,
           be1, We2, be2, W_ih, b_ih, W_hh, b_hh, Wpe, bpe, W1, b1, W2, b2,
           Wo, bo):
    N = node_feats.shape[0]
    G = 256
    NT = Wo.shape[1]
    src = edge_index[0]
    dst = edge_index[1]

    We2b = We2.astype(jnp.bfloat16)
    R = (jnp.arange(H * H)[None, :] // H == jnp.arange(H)[:, None]).astype(
        jnp.bfloat16)

    h = jax.nn.relu(node_feats @ Wp + bp)
    hidden = h
    for _ in range(STEPS):
        hs = jnp.take(h, src, axis=0)
        msg = _msg_pallas(edge_feats, hs, We1, be1, We2b, be2, R)
        agg = jax.ops.segment_sum(msg, dst, num_segments=N)
        m = jax.nn.relu(agg) + h
        gi = m @ W_ih.T + b_ih
        gh = hidden @ W_hh.T + b_hh
        ir, iz, i_n = jnp.split(gi, 3, axis=1)
        hr, hz, hn = jnp.split(gh, 3, axis=1)
        r = jax.nn.sigmoid(ir + hr)
        z = jax.nn.sigmoid(iz + hz)
        n = jnp.tanh(i_n + r * hn)
        hidden = (1.0 - z) * n + z * hidden
        h = hidden

    edge_emb = jax.nn.relu(edge_feats @ Wpe + bpe)
    src_msg = jnp.concatenate([jnp.take(h, src, axis=0), edge_emb], axis=1)
    node_sum = jax.ops.segment_sum(src_msg, dst, num_segments=N)
    mol = jax.ops.segment_sum(node_sum, node_graph_ids, num_segments=G)
    mol = jax.nn.softmax(mol, axis=1)
    x1 = jax.nn.relu(mol @ W1 + b1)
    emb = jax.nn.relu(x1 @ W2 + b2)
    out = emb @ Wo + bo
    logits = out.reshape(-1, NT)
    return jax.nn.sigmoid(logits)
